# Initial kernel scaffold; baseline (speedup 1.0000x reference)
import jax
import jax.numpy as jnp
from jax.experimental import pallas as pl


def kernel(x, edge_index, W, b):
    raise NotImplementedError("write your pallas kernel here")


# speedup vs baseline: 24.3826x; 24.3826x over previous
import functools

import jax
import jax.numpy as jnp
from jax import lax
from jax.experimental import pallas as pl
from jax.experimental.pallas import tpu as pltpu
from jax.experimental.pallas import tpu_sc as plsc

N = 10000
D = 128
HD = D // 2
N_EDGES = 320000

NC = 2
NS = 16
NW = NC * NS

NPAD = 10240
RPT = NPAD // NS

A_CHUNK = 128
A_CHUNKS = -(-N_EDGES // (NW * A_CHUNK))
A_EPAD = NW * A_CHUNKS * A_CHUNK
DEG_W = 16

C_CHUNK = 128
C_CHUNKS = -(-N_EDGES // (NS * C_CHUNK))
C_EPAD = NS * C_CHUNKS * C_CHUNK


def _sc_mesh():
    return plsc.VectorSubcoreMesh(
        core_axis_name="c", subcore_axis_name="s", num_cores=NC, num_subcores=NS
    )


def _deg_body(dst_hbm, ones_hbm, zeros_hbm, deg_hbm, dst_v, ones_v, zeros_v, deg_sh):
    c = lax.axis_index("c")
    s = lax.axis_index("s")
    gid = c * NS + s
    pltpu.sync_copy(dst_hbm.at[gid], dst_v)
    pltpu.sync_copy(ones_hbm, ones_v)
    pltpu.sync_copy(zeros_hbm, zeros_v)
    pltpu.sync_copy(zeros_v, deg_sh.at[pl.ds(s * RPT, RPT)])
    plsc.subcore_barrier()

    @pl.loop(0, A_CHUNKS)
    def _chunk(j):
        pltpu.sync_copy(ones_v, deg_sh.at[dst_v.at[j]], add=True)

    plsc.subcore_barrier()
    pltpu.sync_copy(
        deg_sh.at[pl.ds(s * RPT, RPT)],
        deg_hbm.at[c, pl.ds(s * RPT, RPT)],
    )


@functools.cache
def _deg_kernel():
    return pl.kernel(
        _deg_body,
        out_type=jax.ShapeDtypeStruct((NC, NPAD, DEG_W), jnp.float32),
        mesh=_sc_mesh(),
        compiler_params=pltpu.CompilerParams(use_tc_tiling_on_sc=False),
        scratch_types=[
            pltpu.VMEM((A_CHUNKS, A_CHUNK), jnp.int32),
            pltpu.VMEM((A_CHUNK, DEG_W), jnp.float32),
            pltpu.VMEM((RPT, DEG_W), jnp.float32),
            pltpu.VMEM_SHARED((NPAD, DEG_W), jnp.float32),
        ],
    )


_BM = 512


def _matmul_body(x_ref, w_ref, deg_ref, h2_ref, dinv_ref):
    d = deg_ref[0, :, 0:1] + deg_ref[1, :, 0:1] + 1.0
    di = lax.rsqrt(d)
    h = jnp.dot(x_ref[...], w_ref[0], preferred_element_type=jnp.float32)
    h2_ref[0] = h * di
    dinv_ref[...] = di


def _matmul(x_pad, w, deg):
    return pl.pallas_call(
        _matmul_body,
        grid=(NPAD // _BM, NC),
        in_specs=[
            pl.BlockSpec((_BM, D), lambda i, s: (i, 0)),
            pl.BlockSpec((1, D, HD), lambda i, s: (s, 0, 0)),
            pl.BlockSpec((NC, _BM, DEG_W), lambda i, s: (0, i, 0)),
        ],
        out_specs=[
            pl.BlockSpec((1, _BM, HD), lambda i, s: (s, i, 0)),
            pl.BlockSpec((_BM, 1), lambda i, s: (i, 0)),
        ],
        out_shape=[
            jax.ShapeDtypeStruct((NC, NPAD, HD), jnp.float32),
            jax.ShapeDtypeStruct((NPAD, 1), jnp.float32),
        ],
    )(x_pad, w, deg)


def _scatter_body(h2s_hbm, ei_hbm, acc_hbm,
                  idx_v, rows_v, acc_sh, isem0, isem1, gsem0, gsem1):
    c = lax.axis_index("c")
    s = lax.axis_index("s")
    base = s * RPT
    pltpu.sync_copy(h2s_hbm.at[c, pl.ds(base, RPT)], acc_sh.at[pl.ds(base, RPT)])
    plsc.subcore_barrier()

    isems = [isem0, isem1]
    gsems = [gsem0, gsem1]
    pltpu.sync_copy(ei_hbm.at[s, 0], idx_v.at[0])
    pltpu.async_copy(h2s_hbm.at[c].at[idx_v.at[0, 0]], rows_v.at[0], gsems[0])
    pltpu.async_copy(ei_hbm.at[s, 1], idx_v.at[1], isems[1])

    @pl.loop(0, C_CHUNKS, step=2)
    def _chunk(j):
        for k in range(2):
            jj = j + k

            @pl.when(jj < C_CHUNKS)
            def _():
                pltpu.make_async_copy(
                    h2s_hbm.at[c].at[idx_v.at[k, 0]], rows_v.at[k], gsems[k]
                ).wait()

                @pl.when(jj + 1 < C_CHUNKS)
                def _():
                    pltpu.make_async_copy(
                        ei_hbm.at[s, jj + 1], idx_v.at[1 - k], isems[1 - k]
                    ).wait()
                    pltpu.async_copy(
                        h2s_hbm.at[c].at[idx_v.at[1 - k, 0]],
                        rows_v.at[1 - k],
                        gsems[1 - k],
                    )

                pltpu.sync_copy(rows_v.at[k], acc_sh.at[idx_v.at[k, 1]], add=True)

                @pl.when(jj + 2 < C_CHUNKS)
                def _():
                    pltpu.async_copy(ei_hbm.at[s, jj + 2], idx_v.at[k], isems[k])

    plsc.subcore_barrier()
    pltpu.sync_copy(acc_sh.at[pl.ds(base, RPT)], acc_hbm.at[c, pl.ds(base, RPT)])


@functools.cache
def _scatter_kernel():
    return pl.kernel(
        _scatter_body,
        out_type=jax.ShapeDtypeStruct((NC, NPAD, HD), jnp.float32),
        mesh=_sc_mesh(),
        compiler_params=pltpu.CompilerParams(use_tc_tiling_on_sc=False),
        scratch_types=[
            pltpu.VMEM((2, 2, C_CHUNK), jnp.int32),
            pltpu.VMEM((2, C_CHUNK, HD), jnp.float32),
            pltpu.VMEM_SHARED((NPAD, HD), jnp.float32),
            pltpu.SemaphoreType.DMA,
            pltpu.SemaphoreType.DMA,
            pltpu.SemaphoreType.DMA,
            pltpu.SemaphoreType.DMA,
        ],
    )


_BN = 400


def _final_body(acc_ref, dinv_ref, b_ref, out_ref):
    acc = jnp.concatenate([acc_ref[0], acc_ref[1]], axis=1)
    out_ref[...] = acc * dinv_ref[...] + b_ref[...]


def _final(acc, dinv, b2):
    return pl.pallas_call(
        _final_body,
        grid=(N // _BN,),
        in_specs=[
            pl.BlockSpec((NC, _BN, HD), lambda i: (0, i, 0)),
            pl.BlockSpec((_BN, 1), lambda i: (i, 0)),
            pl.BlockSpec((1, D), lambda i: (0, 0)),
        ],
        out_specs=pl.BlockSpec((_BN, D), lambda i: (i, 0)),
        out_shape=jax.ShapeDtypeStruct((N, D), jnp.float32),
    )(acc, dinv, b2)


def kernel(x, edge_index, W, b):
    src = edge_index[0].astype(jnp.int32)
    dst = edge_index[1].astype(jnp.int32)

    a_pad = A_EPAD - N_EDGES
    dst_a = jnp.concatenate([dst, jnp.full((a_pad,), N, jnp.int32)])
    dst_a = dst_a.reshape(NW, A_CHUNKS, A_CHUNK)

    ones_rows = jnp.ones((A_CHUNK, DEG_W), jnp.float32)
    zero_rows = jnp.zeros((RPT, DEG_W), jnp.float32)
    deg = _deg_kernel()(dst_a, ones_rows, zero_rows)

    x_pad = jnp.pad(x, ((0, NPAD - N), (0, 0)))
    w_split = jnp.stack([W[:, :HD], W[:, HD:]], axis=0)
    h2s, dinv = _matmul(x_pad, w_split, deg)

    c_pad = C_EPAD - N_EDGES
    src_c = jnp.concatenate([src, jnp.zeros((c_pad,), jnp.int32)])
    dst_c = jnp.concatenate([dst, jnp.full((c_pad,), N, jnp.int32)])
    ei_c = jnp.stack(
        [src_c.reshape(NS, C_CHUNKS, C_CHUNK), dst_c.reshape(NS, C_CHUNKS, C_CHUNK)],
        axis=2,
    )
    acc = _scatter_kernel()(h2s, ei_c)

    return _final(acc, dinv, b.reshape(1, D))


# speedup vs baseline: 30.3593x; 1.2451x over previous
import functools

import jax
import jax.numpy as jnp
from jax import lax
from jax.experimental import pallas as pl
from jax.experimental.pallas import tpu as pltpu
from jax.experimental.pallas import tpu_sc as plsc

N = 10000
D = 128
HD = D // 2
N_EDGES = 320000

ACC_DT = jnp.float32

NC = 2
NS = 16
NW = NC * NS

NPAD = 10240
RPT = NPAD // NS

A_CHUNK = 128
A_CHUNKS = -(-N_EDGES // (NW * A_CHUNK))
A_EPAD = NW * A_CHUNKS * A_CHUNK
DEG_W = 16

C_CHUNK = 128
C_CHUNKS = -(-N_EDGES // (NS * C_CHUNK))
C_EPAD = NS * C_CHUNKS * C_CHUNK


def _sc_mesh():
    return plsc.VectorSubcoreMesh(
        core_axis_name="c", subcore_axis_name="s", num_cores=NC, num_subcores=NS
    )


def _deg_body(dst_hbm, ones_hbm, zeros_hbm, deg_hbm, dst_v, ones_v, zeros_v, deg_sh,
              *sems):
    c = lax.axis_index("c")
    s = lax.axis_index("s")
    gid = c * NS + s
    pltpu.sync_copy(dst_hbm.at[gid], dst_v)
    pltpu.sync_copy(ones_hbm, ones_v)
    pltpu.sync_copy(zeros_hbm, zeros_v)
    pltpu.sync_copy(zeros_v, deg_sh.at[pl.ds(s * RPT, RPT)])
    plsc.subcore_barrier()

    def scat(jj, k):
        return pltpu.make_async_copy(ones_v, deg_sh.at[dst_v.at[jj]], sems[k])

    @pl.loop(0, A_CHUNKS, step=NB)
    def _chunk(j):
        for k in range(NB):
            jj = j + k

            @pl.when(jj < A_CHUNKS)
            def _():
                @pl.when(jj >= NB)
                def _():
                    scat(jj - NB, k).wait()

                pltpu.async_copy(ones_v, deg_sh.at[dst_v.at[jj]], sems[k], add=True)

    for jj in range(max(A_CHUNKS - NB, 0), A_CHUNKS):
        scat(jj, jj % NB).wait()

    plsc.subcore_barrier()
    pltpu.sync_copy(
        deg_sh.at[pl.ds(s * RPT, RPT)],
        deg_hbm.at[c, pl.ds(s * RPT, RPT)],
    )


@functools.cache
def _deg_kernel():
    return pl.kernel(
        _deg_body,
        out_type=jax.ShapeDtypeStruct((NC, NPAD, DEG_W), jnp.float32),
        mesh=_sc_mesh(),
        compiler_params=pltpu.CompilerParams(use_tc_tiling_on_sc=False),
        scratch_types=[
            pltpu.VMEM((A_CHUNKS, A_CHUNK), jnp.int32),
            pltpu.VMEM((A_CHUNK, DEG_W), jnp.float32),
            pltpu.VMEM((RPT, DEG_W), jnp.float32),
            pltpu.VMEM_SHARED((NPAD, DEG_W), jnp.float32),
        ] + [pltpu.SemaphoreType.DMA] * NB,
    )


_BM = 512


def _matmul_body(x_ref, w_ref, deg_ref, h2_ref, dinv_ref):
    d = deg_ref[0, :, 0:1] + deg_ref[1, :, 0:1] + 1.0
    di = lax.rsqrt(d)
    h = jnp.dot(x_ref[...], w_ref[0], preferred_element_type=jnp.float32)
    h2_ref[0] = (h * di).astype(ACC_DT)
    dinv_ref[...] = di


def _matmul(x_pad, w, deg):
    return pl.pallas_call(
        _matmul_body,
        grid=(NPAD // _BM, NC),
        in_specs=[
            pl.BlockSpec((_BM, D), lambda i, s: (i, 0)),
            pl.BlockSpec((1, D, HD), lambda i, s: (s, 0, 0)),
            pl.BlockSpec((NC, _BM, DEG_W), lambda i, s: (0, i, 0)),
        ],
        out_specs=[
            pl.BlockSpec((1, _BM, HD), lambda i, s: (s, i, 0)),
            pl.BlockSpec((_BM, 1), lambda i, s: (i, 0)),
        ],
        out_shape=[
            jax.ShapeDtypeStruct((NC, NPAD, HD), ACC_DT),
            jax.ShapeDtypeStruct((NPAD, 1), jnp.float32),
        ],
    )(x_pad, w, deg)


NB = 4
CNB = 4
CL = 3


def _scatter_body(h2s_hbm, ei_hbm, acc_hbm, idx_v, rows_v, acc_sh, *sems):
    gsems = sems[:CNB]
    ssems = sems[CNB:]
    c = lax.axis_index("c")
    s = lax.axis_index("s")
    base = s * RPT
    pltpu.sync_copy(h2s_hbm.at[c, pl.ds(base, RPT)], acc_sh.at[pl.ds(base, RPT)])
    pltpu.sync_copy(ei_hbm.at[s], idx_v)
    plsc.subcore_barrier()

    def gather(jj, k):
        return pltpu.make_async_copy(
            h2s_hbm.at[c].at[idx_v.at[jj, 0]], rows_v.at[k], gsems[k]
        )

    def scatter_wait(jj, k):
        return pltpu.make_async_copy(rows_v.at[k], acc_sh.at[idx_v.at[jj, 1]], ssems[k])

    for k in range(CL):
        gather(k, k).start()

    @pl.loop(0, C_CHUNKS, step=CNB)
    def _chunk(j):
        for k in range(CNB):
            jj = j + k

            @pl.when(jj < C_CHUNKS)
            def _():
                gather(jj, k).wait()
                pltpu.async_copy(
                    rows_v.at[k], acc_sh.at[idx_v.at[jj, 1]], ssems[k], add=True
                )

                @pl.when(jj + CL < C_CHUNKS)
                def _():
                    kp = (k + CL) % CNB
                    @pl.when(jj + CL >= CNB)
                    def _():
                        scatter_wait(jj + CL - CNB, kp).wait()

                    gather(jj + CL, kp).start()

    for jj in range(max(C_CHUNKS - CNB, 0), C_CHUNKS):
        scatter_wait(jj, jj % CNB).wait()

    plsc.subcore_barrier()
    pltpu.sync_copy(acc_sh.at[pl.ds(base, RPT)], acc_hbm.at[c, pl.ds(base, RPT)])


@functools.cache
def _scatter_kernel():
    return pl.kernel(
        _scatter_body,
        out_type=jax.ShapeDtypeStruct((NC, NPAD, HD), ACC_DT),
        mesh=_sc_mesh(),
        compiler_params=pltpu.CompilerParams(use_tc_tiling_on_sc=False),
        scratch_types=[
            pltpu.VMEM((C_CHUNKS, 2, C_CHUNK), jnp.int32),
            pltpu.VMEM((CNB, C_CHUNK, HD), ACC_DT),
            pltpu.VMEM_SHARED((NPAD, HD), ACC_DT),
        ] + [pltpu.SemaphoreType.DMA] * (2 * CNB),
    )


_BN = 400


def _final_body(acc_ref, dinv_ref, b_ref, out_ref):
    acc = jnp.concatenate([acc_ref[0], acc_ref[1]], axis=1).astype(jnp.float32)
    out_ref[...] = acc * dinv_ref[...] + b_ref[...]


def _final(acc, dinv, b2):
    return pl.pallas_call(
        _final_body,
        grid=(N // _BN,),
        in_specs=[
            pl.BlockSpec((NC, _BN, HD), lambda i: (0, i, 0)),
            pl.BlockSpec((_BN, 1), lambda i: (i, 0)),
            pl.BlockSpec((1, D), lambda i: (0, 0)),
        ],
        out_specs=pl.BlockSpec((_BN, D), lambda i: (i, 0)),
        out_shape=jax.ShapeDtypeStruct((N, D), jnp.float32),
    )(acc, dinv, b2)


def kernel(x, edge_index, W, b):
    src = edge_index[0].astype(jnp.int32)
    dst = edge_index[1].astype(jnp.int32)

    a_pad = A_EPAD - N_EDGES
    dst_a = jnp.concatenate([dst, jnp.full((a_pad,), N, jnp.int32)])
    dst_a = dst_a.reshape(NW, A_CHUNKS, A_CHUNK)

    ones_rows = jnp.ones((A_CHUNK, DEG_W), jnp.float32)
    zero_rows = jnp.zeros((RPT, DEG_W), jnp.float32)
    deg = _deg_kernel()(dst_a, ones_rows, zero_rows)

    x_pad = jnp.pad(x, ((0, NPAD - N), (0, 0)))
    w_split = jnp.stack([W[:, :HD], W[:, HD:]], axis=0)
    h2s, dinv = _matmul(x_pad, w_split, deg)

    c_pad = C_EPAD - N_EDGES
    src_c = jnp.concatenate([src, jnp.zeros((c_pad,), jnp.int32)])
    dst_c = jnp.concatenate([dst, jnp.full((c_pad,), N, jnp.int32)])
    ei_c = jnp.stack(
        [src_c.reshape(NS, C_CHUNKS, C_CHUNK), dst_c.reshape(NS, C_CHUNKS, C_CHUNK)],
        axis=2,
    )
    acc = _scatter_kernel()(h2s, ei_c)

    return _final(acc, dinv, b.reshape(1, D))


# speedup vs baseline: 30.8525x; 1.0162x over previous
import functools

import jax
import jax.numpy as jnp
from jax import lax
from jax.experimental import pallas as pl
from jax.experimental.pallas import tpu as pltpu
from jax.experimental.pallas import tpu_sc as plsc

N = 10000
D = 128
HD = D // 2
N_EDGES = 320000

ACC_DT = jnp.float32

NC = 2
NS = 16
NW = NC * NS

NPAD = 10240
RPT = NPAD // NS

A_CHUNK = 128
A_CHUNKS = -(-N_EDGES // (NW * A_CHUNK))
A_EPAD = NW * A_CHUNKS * A_CHUNK
DEG_W = 8

C_CHUNK = 128
C_CHUNKS = -(-N_EDGES // (NS * C_CHUNK))
C_EPAD = NS * C_CHUNKS * C_CHUNK


def _sc_mesh():
    return plsc.VectorSubcoreMesh(
        core_axis_name="c", subcore_axis_name="s", num_cores=NC, num_subcores=NS
    )


def _deg_body(dst_hbm, ones_hbm, zeros_hbm, deg_hbm, dst_v, ones_v, zeros_v, deg_sh,
              *sems):
    c = lax.axis_index("c")
    s = lax.axis_index("s")
    gid = c * NS + s
    pltpu.sync_copy(dst_hbm.at[gid], dst_v)
    pltpu.sync_copy(ones_hbm, ones_v)
    pltpu.sync_copy(zeros_hbm, zeros_v)
    pltpu.sync_copy(zeros_v, deg_sh.at[pl.ds(s * RPT, RPT)])
    plsc.subcore_barrier()

    def scat(jj, k):
        return pltpu.make_async_copy(ones_v, deg_sh.at[dst_v.at[jj]], sems[k])

    @pl.loop(0, A_CHUNKS, step=NB)
    def _chunk(j):
        for k in range(NB):
            jj = j + k

            @pl.when(jj < A_CHUNKS)
            def _():
                @pl.when(jj >= NB)
                def _():
                    scat(jj - NB, k).wait()

                pltpu.async_copy(ones_v, deg_sh.at[dst_v.at[jj]], sems[k], add=True)

    for jj in range(max(A_CHUNKS - NB, 0), A_CHUNKS):
        scat(jj, jj % NB).wait()

    plsc.subcore_barrier()
    pltpu.sync_copy(
        deg_sh.at[pl.ds(s * RPT, RPT)],
        deg_hbm.at[c, pl.ds(s * RPT, RPT)],
    )


@functools.cache
def _deg_kernel():
    return pl.kernel(
        _deg_body,
        out_type=jax.ShapeDtypeStruct((NC, NPAD, DEG_W), jnp.float32),
        mesh=_sc_mesh(),
        compiler_params=pltpu.CompilerParams(use_tc_tiling_on_sc=False),
        scratch_types=[
            pltpu.VMEM((A_CHUNKS, A_CHUNK), jnp.int32),
            pltpu.VMEM((A_CHUNK, DEG_W), jnp.float32),
            pltpu.VMEM((RPT, DEG_W), jnp.float32),
            pltpu.VMEM_SHARED((NPAD, DEG_W), jnp.float32),
        ] + [pltpu.SemaphoreType.DMA] * NB,
    )


_BM = 512


def _matmul_body(x_ref, w_ref, deg_ref, h2_ref, dinv_ref):
    d = deg_ref[0, :, 0:1] + deg_ref[1, :, 0:1] + 1.0
    di = lax.rsqrt(d)
    h = jnp.dot(x_ref[...], w_ref[0], preferred_element_type=jnp.float32)
    h2_ref[0] = (h * di).astype(ACC_DT)
    dinv_ref[...] = di


def _matmul(x_pad, w, deg):
    return pl.pallas_call(
        _matmul_body,
        grid=(NPAD // _BM, NC),
        in_specs=[
            pl.BlockSpec((_BM, D), lambda i, s: (i, 0)),
            pl.BlockSpec((1, D, HD), lambda i, s: (s, 0, 0)),
            pl.BlockSpec((NC, _BM, DEG_W), lambda i, s: (0, i, 0)),
        ],
        out_specs=[
            pl.BlockSpec((1, _BM, HD), lambda i, s: (s, i, 0)),
            pl.BlockSpec((_BM, 1), lambda i, s: (i, 0)),
        ],
        out_shape=[
            jax.ShapeDtypeStruct((NC, NPAD, HD), ACC_DT),
            jax.ShapeDtypeStruct((NPAD, 1), jnp.float32),
        ],
    )(x_pad, w, deg)


NB = 4
CNB = 4
CL = 3


def _scatter_body(h2s_hbm, ei_hbm, acc_hbm, idx_v, rows_v, acc_sh, *sems):
    gsems = sems[:CNB]
    ssems = sems[CNB:]
    c = lax.axis_index("c")
    s = lax.axis_index("s")
    base = s * RPT
    pltpu.sync_copy(h2s_hbm.at[c, pl.ds(base, RPT)], acc_sh.at[pl.ds(base, RPT)])
    pltpu.sync_copy(ei_hbm.at[s], idx_v)
    plsc.subcore_barrier()

    def gather(jj, k):
        return pltpu.make_async_copy(
            h2s_hbm.at[c].at[idx_v.at[jj, 0]], rows_v.at[k], gsems[k]
        )

    def scatter_wait(jj, k):
        return pltpu.make_async_copy(rows_v.at[k], acc_sh.at[idx_v.at[jj, 1]], ssems[k])

    for k in range(CL):
        gather(k, k).start()

    @pl.loop(0, C_CHUNKS, step=CNB)
    def _chunk(j):
        for k in range(CNB):
            jj = j + k

            @pl.when(jj < C_CHUNKS)
            def _():
                gather(jj, k).wait()
                pltpu.async_copy(
                    rows_v.at[k], acc_sh.at[idx_v.at[jj, 1]], ssems[k], add=True
                )

                @pl.when(jj + CL < C_CHUNKS)
                def _():
                    kp = (k + CL) % CNB
                    @pl.when(jj + CL >= CNB)
                    def _():
                        scatter_wait(jj + CL - CNB, kp).wait()

                    gather(jj + CL, kp).start()

    for jj in range(max(C_CHUNKS - CNB, 0), C_CHUNKS):
        scatter_wait(jj, jj % CNB).wait()

    plsc.subcore_barrier()
    pltpu.sync_copy(acc_sh.at[pl.ds(base, RPT)], acc_hbm.at[c, pl.ds(base, RPT)])


@functools.cache
def _scatter_kernel():
    return pl.kernel(
        _scatter_body,
        out_type=jax.ShapeDtypeStruct((NC, NPAD, HD), ACC_DT),
        mesh=_sc_mesh(),
        compiler_params=pltpu.CompilerParams(use_tc_tiling_on_sc=False),
        scratch_types=[
            pltpu.VMEM((C_CHUNKS, 2, C_CHUNK), jnp.int32),
            pltpu.VMEM((CNB, C_CHUNK, HD), ACC_DT),
            pltpu.VMEM_SHARED((NPAD, HD), ACC_DT),
        ] + [pltpu.SemaphoreType.DMA] * (2 * CNB),
    )


_BN = 400


def _final_body(acc_ref, dinv_ref, b_ref, out_ref):
    acc = jnp.concatenate([acc_ref[0], acc_ref[1]], axis=1).astype(jnp.float32)
    out_ref[...] = acc * dinv_ref[...] + b_ref[...]


def _final(acc, dinv, b2):
    return pl.pallas_call(
        _final_body,
        grid=(N // _BN,),
        in_specs=[
            pl.BlockSpec((NC, _BN, HD), lambda i: (0, i, 0)),
            pl.BlockSpec((_BN, 1), lambda i: (i, 0)),
            pl.BlockSpec((1, D), lambda i: (0, 0)),
        ],
        out_specs=pl.BlockSpec((_BN, D), lambda i: (i, 0)),
        out_shape=jax.ShapeDtypeStruct((N, D), jnp.float32),
    )(acc, dinv, b2)


def kernel(x, edge_index, W, b):
    src = edge_index[0].astype(jnp.int32)
    dst = edge_index[1].astype(jnp.int32)

    a_pad = A_EPAD - N_EDGES
    dst_a = jnp.concatenate([dst, jnp.full((a_pad,), N, jnp.int32)])
    dst_a = dst_a.reshape(NW, A_CHUNKS, A_CHUNK)

    ones_rows = jnp.ones((A_CHUNK, DEG_W), jnp.float32)
    zero_rows = jnp.zeros((RPT, DEG_W), jnp.float32)
    deg = _deg_kernel()(dst_a, ones_rows, zero_rows)

    x_pad = jnp.pad(x, ((0, NPAD - N), (0, 0)))
    w_split = jnp.stack([W[:, :HD], W[:, HD:]], axis=0)
    h2s, dinv = _matmul(x_pad, w_split, deg)

    c_pad = C_EPAD - N_EDGES
    src_c = jnp.concatenate([src, jnp.zeros((c_pad,), jnp.int32)])
    dst_c = jnp.concatenate([dst, jnp.full((c_pad,), N, jnp.int32)])
    ei_c = jnp.stack(
        [src_c.reshape(NS, C_CHUNKS, C_CHUNK), dst_c.reshape(NS, C_CHUNKS, C_CHUNK)],
        axis=2,
    )
    acc = _scatter_kernel()(h2s, ei_c)

    return _final(acc, dinv, b.reshape(1, D))
